# baseline (speedup 1.0000x reference)
import jax
import jax.numpy as jnp
from jax.experimental import pallas as pl
from jax.experimental.pallas import tpu as pltpu

BETA_P1 = 1.25


def _vq_body(x_ref, cb_ref, cbt_ref, q_ref, loss_ref):
    b = pl.program_id(0)
    cb = cb_ref[...]
    C = cb_ref.shape[0]
    G = x_ref.shape[0]

    cb_sq = jnp.sum(cb * cb, axis=1, keepdims=True)
    iota_i = jax.lax.broadcasted_iota(jnp.int32, (C, 1), 0)
    iota_c = iota_i.astype(jnp.float32)
    iota_16 = iota_i.astype(jnp.int16)
    cbt_bf = cbt_ref[...].astype(jnp.bfloat16)

    @pl.when(b == 0)
    def _():
        loss_ref[0, 0] = 0.0

    mms = [
        jax.lax.dot_general(
            cb, x_ref[g], (((1,), (0,)), ((), ())),
            preferred_element_type=jnp.float32)
        for g in range(G)
    ]
    for g in range(G):
        xb = x_ref[g]
        x_sq = jnp.sum(xb * xb, axis=0, keepdims=True)
        mm = mms[g]
        dist = (0.5 * x_sq + 0.5 * cb_sq) - mm

        minv = jnp.min(dist, axis=0, keepdims=True)
        masked = jnp.where(dist == minv, iota_c, jnp.float32(C))
        idx = jnp.min(masked, axis=0, keepdims=True)
        idx16 = idx.astype(jnp.int32).astype(jnp.int16)
        onehot = jnp.where(iota_16 == idx16,
                           jnp.bfloat16(1), jnp.bfloat16(0))

        q = jax.lax.dot_general(
            cbt_bf, onehot, (((1,), (0,)), ((), ())),
            preferred_element_type=jnp.float32)
        q_ref[g] = q
        loss_ref[0, 0] += jnp.sum(minv)


def kernel(latents, codebook):
    B, D, H, W = latents.shape
    C = codebook.shape[0]
    HW = H * W
    x = latents.reshape(B, D, HW)
    cbt = codebook.T

    G = 4
    q, s = pl.pallas_call(
        _vq_body,
        grid=(B // G,),
        in_specs=[
            pl.BlockSpec((G, D, HW), lambda b: (b, 0, 0)),
            pl.BlockSpec((C, D), lambda b: (0, 0)),
            pl.BlockSpec((D, C), lambda b: (0, 0)),
        ],
        out_specs=[
            pl.BlockSpec((G, D, HW), lambda b: (b, 0, 0)),
            pl.BlockSpec(memory_space=pltpu.SMEM),
        ],
        out_shape=[
            jax.ShapeDtypeStruct((B, D, HW), jnp.float32),
            jax.ShapeDtypeStruct((1, 1), jnp.float32),
        ],
    )(x, codebook, cbt)

    vq_loss = (2.0 * BETA_P1 / (B * HW * D)) * s[0, 0]
    return (q.reshape(B, D, H, W), vq_loss)


# speedup vs baseline: 1.0099x; 1.0099x over previous
import jax
import jax.numpy as jnp
from jax.experimental import pallas as pl
from jax.experimental.pallas import tpu as pltpu

BETA_P1 = 1.25


def _vq_body(x_ref, cb_ref, cbt_ref, q_ref, loss_ref):
    b = pl.program_id(0)
    cb = cb_ref[...]
    C = cb_ref.shape[0]
    G = x_ref.shape[0]

    cb_sq = jnp.sum(cb * cb, axis=1, keepdims=True)
    iota_c = jax.lax.broadcasted_iota(jnp.int32, (C, 1), 0).astype(jnp.float32)

    @pl.when(b == 0)
    def _():
        loss_ref[0, 0] = 0.0

    mms = [
        jax.lax.dot_general(
            cb, x_ref[g], (((1,), (0,)), ((), ())),
            preferred_element_type=jnp.float32)
        for g in range(G)
    ]
    for g in range(G):
        xb = x_ref[g]
        x_sq = jnp.sum(xb * xb, axis=0, keepdims=True)
        mm = mms[g]
        dist = (0.5 * x_sq + 0.5 * cb_sq) - mm

        minv = jnp.min(dist, axis=0, keepdims=True)
        masked = jnp.where(dist == minv, iota_c, jnp.float32(C))
        idx = jnp.min(masked, axis=0, keepdims=True)
        onehot = (iota_c == idx).astype(jnp.float32)

        q = jax.lax.dot_general(
            cbt_ref[...], onehot, (((1,), (0,)), ((), ())),
            preferred_element_type=jnp.float32)
        q_ref[g] = q
        loss_ref[0, 0] += jnp.sum(minv)


def kernel(latents, codebook):
    B, D, H, W = latents.shape
    C = codebook.shape[0]
    HW = H * W
    x = latents.reshape(B, D, HW)
    cbt = codebook.T

    G = 4
    q, s = pl.pallas_call(
        _vq_body,
        grid=(B // G,),
        in_specs=[
            pl.BlockSpec((G, D, HW), lambda b: (b, 0, 0)),
            pl.BlockSpec((C, D), lambda b: (0, 0)),
            pl.BlockSpec((D, C), lambda b: (0, 0)),
        ],
        out_specs=[
            pl.BlockSpec((G, D, HW), lambda b: (b, 0, 0)),
            pl.BlockSpec(memory_space=pltpu.SMEM),
        ],
        out_shape=[
            jax.ShapeDtypeStruct((B, D, HW), jnp.float32),
            jax.ShapeDtypeStruct((1, 1), jnp.float32),
        ],
    )(x, codebook, cbt)

    vq_loss = (2.0 * BETA_P1 / (B * HW * D)) * s[0, 0]
    return (q.reshape(B, D, H, W), vq_loss)
